# baseline (speedup 1.0000x reference)
import functools

import jax
import jax.numpy as jnp
from jax import lax
from jax.experimental import pallas as pl
from jax.experimental.pallas import tpu as pltpu
from jax.experimental.pallas import tpu_sc as plsc

B, S, F = 32, 2048, 512
SEG = 256
LANES = 16


def _cutmix_sc(x, indices, starts):
    mesh = plsc.VectorSubcoreMesh(core_axis_name="c", subcore_axis_name="s")
    info = plsc.get_sparse_core_info()
    nc = info.num_cores

    @functools.partial(
        pl.kernel,
        mesh=mesh,
        out_type=jax.ShapeDtypeStruct((B, S, F), jnp.float32),
        scratch_types=[
            pltpu.VMEM((B + 16,), jnp.int32),
            pltpu.VMEM((B + 16,), jnp.int32),
            pltpu.VMEM((8, F), jnp.float32),
            pltpu.VMEM((8, F), jnp.float32),
        ],
    )
    def k(x_hbm, idx_hbm, st_hbm, out_hbm, idx_v, st_v, buf_i, buf_p):
        wid = lax.axis_index("s") * nc + lax.axis_index("c")
        pltpu.sync_copy(idx_hbm, idx_v.at[pl.ds(0, B)])
        pltpu.sync_copy(st_hbm, st_v.at[pl.ds(0, B)])
        p = idx_v[pl.ds(wid, LANES)][0]
        st = st_v[pl.ds(wid, LANES)][0]
        m = lax.rem(st, 8)

        pltpu.sync_copy(x_hbm.at[wid], out_hbm.at[wid])

        @pl.when(m == 0)
        def _aligned():
            st_a = pl.multiple_of(st, 8)
            pltpu.sync_copy(
                x_hbm.at[p, pl.ds(st_a, SEG)], out_hbm.at[wid, pl.ds(st_a, SEG)]
            )

        @pl.when(m != 0)
        def _unaligned():
            a0 = pl.multiple_of(st - m, 8)
            b0 = pl.multiple_of(st + SEG - m, 8)
            i0 = pl.multiple_of(st - m + 8, 8)
            pltpu.sync_copy(
                x_hbm.at[p, pl.ds(i0, SEG - 8)],
                out_hbm.at[wid, pl.ds(i0, SEG - 8)],
            )

            def merge_edge(base, from_p):
                pltpu.sync_copy(x_hbm.at[wid, pl.ds(base, 8)], buf_i)
                pltpu.sync_copy(x_hbm.at[p, pl.ds(base, 8)], buf_p)
                for r in range(8):
                    @pl.when(from_p(r))
                    def _row():
                        for c in range(F // LANES):
                            sl = pl.ds(c * LANES, LANES)
                            buf_i[r, sl] = buf_p[r, sl]
                pltpu.sync_copy(buf_i, out_hbm.at[wid, pl.ds(base, 8)])

            merge_edge(a0, lambda r: r >= m)
            merge_edge(b0, lambda r: r < m)

    return k(x, indices, starts)


def kernel(x, indices, starts):
    return _cutmix_sc(x, indices, starts)


# speedup vs baseline: 34.6445x; 34.6445x over previous
import functools

import jax
import jax.numpy as jnp
from jax import lax
from jax.experimental import pallas as pl
from jax.experimental.pallas import tpu as pltpu
from jax.experimental.pallas import tpu_sc as plsc

B, S, F = 32, 2048, 512
SEG = 256
LANES = 16
CH = 64
NCH = S // CH


def _cutmix_sc(x, indices, starts):
    mesh = plsc.VectorSubcoreMesh(core_axis_name="c", subcore_axis_name="s")
    info = plsc.get_sparse_core_info()
    nc = info.num_cores

    @functools.partial(
        pl.kernel,
        mesh=mesh,
        out_type=jax.ShapeDtypeStruct((B, S, F), jnp.float32),
        scratch_types=[
            pltpu.VMEM((B + 16,), jnp.int32),
            pltpu.VMEM((B + 16,), jnp.int32),
            pltpu.VMEM((CH, F), jnp.float32),
            pltpu.VMEM((CH, F), jnp.float32),
            pltpu.VMEM((8, F), jnp.float32),
            pltpu.VMEM((8, F), jnp.float32),
            pltpu.SemaphoreType.DMA,
            pltpu.SemaphoreType.DMA,
            pltpu.SemaphoreType.DMA,
            pltpu.SemaphoreType.DMA,
        ],
    )
    def k(x_hbm, idx_hbm, st_hbm, out_hbm, idx_v, st_v,
          buf0, buf1, buf_i, buf_p, si0, si1, so0, so1):
        wid = lax.axis_index("s") * nc + lax.axis_index("c")
        pltpu.sync_copy(idx_hbm, idx_v.at[pl.ds(0, B)])
        pltpu.sync_copy(st_hbm, st_v.at[pl.ds(0, B)])
        p = idx_v[pl.ds(wid, LANES)][0]
        st = st_v[pl.ds(wid, LANES)][0]
        m = lax.rem(st, 8)

        bufs = (buf0, buf1)
        sin = (si0, si1)
        sout = (so0, so1)

        def start_in(c):
            b = c % 2
            return pltpu.async_copy(
                x_hbm.at[wid, pl.ds(c * CH, CH)], bufs[b], sin[b])

        def start_out(c):
            b = c % 2
            return pltpu.async_copy(
                bufs[b], out_hbm.at[wid, pl.ds(c * CH, CH)], sout[b])

        in_h = [None, None]
        out_h = [None, None]
        in_h[0] = start_in(0)
        for c in range(NCH):
            b = c % 2
            if c + 1 < NCH:
                b2 = (c + 1) % 2
                if c >= 1:
                    out_h[b2].wait()
                in_h[b2] = start_in(c + 1)
            in_h[b].wait()
            out_h[b] = start_out(c)
        out_h[0].wait()
        out_h[1].wait()

        def seg_copy(src_off, dst_off, n):
            pltpu.sync_copy(
                x_hbm.at[p, pl.ds(src_off, n)], buf0.at[pl.ds(0, n)])
            pltpu.sync_copy(
                buf0.at[pl.ds(0, n)], out_hbm.at[wid, pl.ds(dst_off, n)])

        @pl.when(m == 0)
        def _aligned():
            st_a = pl.multiple_of(st, 8)
            for j in range(SEG // CH):
                seg_copy(st_a + j * CH, st_a + j * CH, CH)

        @pl.when(m != 0)
        def _unaligned():
            a0 = pl.multiple_of(st - m, 8)
            b0 = pl.multiple_of(st + SEG - m, 8)
            for j, n in ((0, CH), (1, CH), (2, CH), (3, CH - 8)):
                off = pl.multiple_of(a0 + 8 + j * CH, 8)
                seg_copy(off, off, n)

            def merge_edge(base, from_p):
                pltpu.sync_copy(x_hbm.at[wid, pl.ds(base, 8)], buf_i)
                pltpu.sync_copy(x_hbm.at[p, pl.ds(base, 8)], buf_p)
                for r in range(8):
                    @pl.when(from_p(r))
                    def _row():
                        for c in range(F // LANES):
                            sl = pl.ds(c * LANES, LANES)
                            buf_i[r, sl] = buf_p[r, sl]
                pltpu.sync_copy(buf_i, out_hbm.at[wid, pl.ds(base, 8)])

            merge_edge(a0, lambda r: r >= m)
            merge_edge(b0, lambda r: r < m)

    return k(x, indices, starts)


def kernel(x, indices, starts):
    return _cutmix_sc(x, indices, starts)


# speedup vs baseline: 37.1442x; 1.0722x over previous
import functools

import jax
import jax.numpy as jnp
from jax import lax
from jax.experimental import pallas as pl
from jax.experimental.pallas import tpu as pltpu
from jax.experimental.pallas import tpu_sc as plsc

B, S, F = 32, 2048, 512
SEG = 256
LANES = 16
CH = 64
NCH = S // CH
NBUF = 3


def _cutmix_sc(x, indices, starts):
    mesh = plsc.VectorSubcoreMesh(core_axis_name="c", subcore_axis_name="s")
    info = plsc.get_sparse_core_info()
    nc = info.num_cores

    @functools.partial(
        pl.kernel,
        mesh=mesh,
        out_type=jax.ShapeDtypeStruct((B, S, F), jnp.float32),
        scratch_types=[
            pltpu.VMEM((B + 16,), jnp.int32),
            pltpu.VMEM((B + 16,), jnp.int32),
            pltpu.VMEM((CH, F), jnp.float32),
            pltpu.VMEM((CH, F), jnp.float32),
            pltpu.VMEM((CH, F), jnp.float32),
            pltpu.VMEM((8, F), jnp.float32),
            pltpu.VMEM((8, F), jnp.float32),
            pltpu.SemaphoreType.DMA,
            pltpu.SemaphoreType.DMA,
            pltpu.SemaphoreType.DMA,
            pltpu.SemaphoreType.DMA,
            pltpu.SemaphoreType.DMA,
            pltpu.SemaphoreType.DMA,
        ],
    )
    def k(x_hbm, idx_hbm, st_hbm, out_hbm, idx_v, st_v,
          buf0, buf1, buf2, buf_i, buf_p, si0, si1, si2, so0, so1, so2):
        wid = lax.axis_index("s") * nc + lax.axis_index("c")
        pltpu.sync_copy(idx_hbm, idx_v.at[pl.ds(0, B)])
        pltpu.sync_copy(st_hbm, st_v.at[pl.ds(0, B)])
        p = idx_v[pl.ds(wid, LANES)][0]
        st = st_v[pl.ds(wid, LANES)][0]
        m = lax.rem(st, 8)
        q = lax.rem(st, CH)
        g = (q - m) // 8

        bufs = (buf0, buf1, buf2)
        sin = (si0, si1, si2)
        sout = (so0, so1, so2)

        def start_in(c):
            b = c % NBUF
            c0 = c * CH
            inside = jnp.logical_and(st <= c0, c0 + CH <= st + SEG)
            src = lax.select(inside, p, wid)
            return pltpu.async_copy(
                x_hbm.at[src, pl.ds(c0, CH)], bufs[b], sin[b])

        def start_out(c):
            b = c % NBUF
            return pltpu.async_copy(
                bufs[b], out_hbm.at[wid, pl.ds(c * CH, CH)], sout[b])

        in_h = [None] * NBUF
        out_h = [None] * NBUF
        for c in range(NBUF - 1):
            in_h[c] = start_in(c)
        for c in range(NCH):
            b = c % NBUF
            if c + NBUF - 1 < NCH:
                b2 = (c + NBUF - 1) % NBUF
                if c >= 1:
                    out_h[b2].wait()
                in_h[b2] = start_in(c + NBUF - 1)
            in_h[b].wait()
            out_h[b] = start_out(c)
        for b in range(NBUF):
            out_h[b].wait()

        def seg_copy(off, n):
            pltpu.sync_copy(
                x_hbm.at[p, pl.ds(off, n)], buf0.at[pl.ds(0, n)])
            pltpu.sync_copy(
                buf0.at[pl.ds(0, n)], out_hbm.at[wid, pl.ds(off, n)])

        def copy_8blocks(off, nblocks):
            for j in range(1, CH // 8):
                @pl.when(nblocks == j)
                def _arm():
                    seg_copy(pl.multiple_of(off, 8), 8 * j)

        def merge_edge(base, from_p):
            pltpu.sync_copy(x_hbm.at[wid, pl.ds(base, 8)], buf_i)
            pltpu.sync_copy(x_hbm.at[p, pl.ds(base, 8)], buf_p)
            for r in range(8):
                @pl.when(from_p(r))
                def _row():
                    for c in range(F // LANES):
                        sl = pl.ds(c * LANES, LANES)
                        buf_i[r, sl] = buf_p[r, sl]
            pltpu.sync_copy(buf_i, out_hbm.at[wid, pl.ds(base, 8)])

        @pl.when(jnp.logical_and(q != 0, m == 0))
        def _aligned8():
            copy_8blocks(st, (CH - q) // 8)
            copy_8blocks(st + SEG - q, g)

        @pl.when(m != 0)
        def _unaligned():
            a0 = pl.multiple_of(st - m, 8)
            b0 = pl.multiple_of(st + SEG - m, 8)
            copy_8blocks(a0 + 8, (CH - 8) // 8 - g)
            copy_8blocks(st + SEG - q, g)
            merge_edge(a0, lambda r: r >= m)
            merge_edge(b0, lambda r: r < m)

    return k(x, indices, starts)


def kernel(x, indices, starts):
    return _cutmix_sc(x, indices, starts)


# speedup vs baseline: 37.6010x; 1.0123x over previous
import functools

import jax
import jax.numpy as jnp
from jax import lax
from jax.experimental import pallas as pl
from jax.experimental.pallas import tpu as pltpu
from jax.experimental.pallas import tpu_sc as plsc

B, S, F = 32, 2048, 512
SEG = 256
LANES = 16
CH = 32
NCH = S // CH
NBUF = 6


def _cutmix_sc(x, indices, starts):
    mesh = plsc.VectorSubcoreMesh(core_axis_name="c", subcore_axis_name="s")
    info = plsc.get_sparse_core_info()
    nc = info.num_cores

    @functools.partial(
        pl.kernel,
        mesh=mesh,
        out_type=jax.ShapeDtypeStruct((B, S, F), jnp.float32),
        scratch_types=(
            [pltpu.VMEM((B + 16,), jnp.int32)] * 2
            + [pltpu.VMEM((CH, F), jnp.float32)] * NBUF
            + [pltpu.VMEM((8, F), jnp.float32)] * 2
            + [pltpu.SemaphoreType.DMA] * (2 * NBUF)
        ),
    )
    def k(x_hbm, idx_hbm, st_hbm, out_hbm, *scr):
        idx_v, st_v = scr[0], scr[1]
        bufs = scr[2:2 + NBUF]
        buf_i, buf_p = scr[2 + NBUF], scr[3 + NBUF]
        sin = scr[4 + NBUF:4 + 2 * NBUF]
        sout = scr[4 + 2 * NBUF:4 + 3 * NBUF]

        wid = lax.axis_index("s") * nc + lax.axis_index("c")
        pltpu.sync_copy(idx_hbm, idx_v.at[pl.ds(0, B)])
        pltpu.sync_copy(st_hbm, st_v.at[pl.ds(0, B)])
        p = idx_v[pl.ds(wid, LANES)][0]
        st = st_v[pl.ds(wid, LANES)][0]
        m = lax.rem(st, 8)
        q = lax.rem(st, CH)
        g = (q - m) // 8

        def start_in(c):
            b = c % NBUF
            c0 = c * CH
            inside = jnp.logical_and(st <= c0, c0 + CH <= st + SEG)
            src = lax.select(inside, p, wid)
            return pltpu.async_copy(
                x_hbm.at[src, pl.ds(c0, CH)], bufs[b], sin[b])

        def start_out(c):
            b = c % NBUF
            return pltpu.async_copy(
                bufs[b], out_hbm.at[wid, pl.ds(c * CH, CH)], sout[b])

        in_h = [None] * NBUF
        out_h = [None] * NBUF
        for c in range(NBUF - 1):
            in_h[c] = start_in(c)
        for c in range(NCH):
            b = c % NBUF
            if c + NBUF - 1 < NCH:
                b2 = (c + NBUF - 1) % NBUF
                if c >= 1:
                    out_h[b2].wait()
                in_h[b2] = start_in(c + NBUF - 1)
            in_h[b].wait()
            out_h[b] = start_out(c)
        for b in range(min(NBUF, NCH)):
            out_h[b].wait()

        def seg_copy(off, n):
            pltpu.sync_copy(
                x_hbm.at[p, pl.ds(off, n)], bufs[0].at[pl.ds(0, n)])
            pltpu.sync_copy(
                bufs[0].at[pl.ds(0, n)], out_hbm.at[wid, pl.ds(off, n)])

        def copy_8blocks(off, nblocks):
            for j in range(1, CH // 8):
                @pl.when(nblocks == j)
                def _arm():
                    seg_copy(pl.multiple_of(off, 8), 8 * j)

        def merge_edge(base, from_p):
            pltpu.sync_copy(x_hbm.at[wid, pl.ds(base, 8)], buf_i)
            pltpu.sync_copy(x_hbm.at[p, pl.ds(base, 8)], buf_p)
            for r in range(8):
                @pl.when(from_p(r))
                def _row():
                    for c in range(F // LANES):
                        sl = pl.ds(c * LANES, LANES)
                        buf_i[r, sl] = buf_p[r, sl]
            pltpu.sync_copy(buf_i, out_hbm.at[wid, pl.ds(base, 8)])

        @pl.when(jnp.logical_and(q != 0, m == 0))
        def _aligned8():
            copy_8blocks(st, (CH - q) // 8)
            copy_8blocks(st + SEG - q, g)

        @pl.when(m != 0)
        def _unaligned():
            a0 = pl.multiple_of(st - m, 8)
            b0 = pl.multiple_of(st + SEG - m, 8)
            copy_8blocks(a0 + 8, (CH - 8) // 8 - g)
            copy_8blocks(st + SEG - q, g)
            merge_edge(a0, lambda r: r >= m)
            merge_edge(b0, lambda r: r < m)

    return k(x, indices, starts)


def kernel(x, indices, starts):
    return _cutmix_sc(x, indices, starts)
